# baseline (speedup 1.0000x reference)
import functools

import jax
import jax.numpy as jnp
from jax import lax
from jax.experimental import pallas as pl
from jax.experimental.pallas import tpu as pltpu
from jax.experimental.pallas import tpu_sc as plsc

N = 10000
E = 320000
D1 = 128
D2 = 64
NC = 2
NS = 16
NW = NC * NS
NPAD = 10240
NR = NPAD // NS
EW = 128
EP = 7680
EC = (E + EP) // EW
ET = EC // NW
IG = 16
ZR = 32
BM = 1000

_mesh = plsc.VectorSubcoreMesh(core_axis_name="c", subcore_axis_name="s")


@functools.partial(
    pl.kernel,
    out_type=jax.ShapeDtypeStruct((NC * NPAD,), jnp.float32),
    mesh=_mesh,
    scratch_types=[
        pltpu.VMEM((ET, EW), jnp.int32),
        pltpu.VMEM((EW,), jnp.float32),
        pltpu.VMEM((NR,), jnp.float32),
        pltpu.SemaphoreType.DMA,
        pltpu.VMEM_SHARED((NPAD,), jnp.float32),
    ],
)
def _deg(dst_hbm, out_hbm, di_v, ones_v, zero_v, sem, acc):
    cid = lax.axis_index("c")
    sid = lax.axis_index("s")
    wid = cid * NS + sid

    pltpu.async_copy(dst_hbm.at[pl.ds(wid * ET, ET)], di_v, sem)

    def _fill(k, _):
        zero_v[pl.ds(k * 16, 16)] = jnp.zeros((16,), jnp.float32)
        return 0

    lax.fori_loop(0, NR // 16, _fill, 0)

    def _fill1(k, _):
        ones_v[pl.ds(k * 16, 16)] = jnp.ones((16,), jnp.float32)
        return 0

    lax.fori_loop(0, EW // 16, _fill1, 0)
    pltpu.sync_copy(zero_v, acc.at[pl.ds(sid * NR, NR)])
    pltpu.make_async_copy(dst_hbm.at[pl.ds(wid * ET, ET)], di_v, sem).wait()
    plsc.subcore_barrier()

    def _scat(j, _):
        pltpu.sync_copy(ones_v, acc.at[di_v.at[j]], add=True)
        return 0

    lax.fori_loop(0, ET, _scat, 0)
    plsc.subcore_barrier()
    pltpu.sync_copy(acc.at[pl.ds(sid * NR, NR)],
                    out_hbm.at[pl.ds(cid * NPAD + sid * NR, NR)])


def _make_agg(D):
    @functools.partial(
        pl.kernel,
        out_type=jax.ShapeDtypeStruct((NC, NPAD, D), jnp.float32),
        mesh=_mesh,
        scratch_types=[
            pltpu.VMEM((IG, EW), jnp.int32),
            pltpu.VMEM((IG, EW), jnp.int32),
            pltpu.VMEM((IG, EW), jnp.int32),
            pltpu.VMEM((IG, EW), jnp.int32),
            pltpu.VMEM((EW, D), jnp.float32),
            pltpu.VMEM((EW, D), jnp.float32),
            pltpu.VMEM((ZR, D), jnp.float32),
            pltpu.SemaphoreType.DMA,
            pltpu.SemaphoreType.DMA,
            pltpu.SemaphoreType.DMA,
            pltpu.SemaphoreType.DMA,
            pltpu.VMEM_SHARED((NPAD, D), jnp.float32),
        ],
    )
    def _agg(y_hbm, src_hbm, dst_hbm, out_hbm,
             si_a, si_b, di_a, di_b, r0, r1, zb,
             sem0, sem1, sem_s, sem_d, acc):
        cid = lax.axis_index("c")
        sid = lax.axis_index("s")
        wid = cid * NS + sid

        def _zrow(i, _):
            def _zcol(j, _):
                zb[i, pl.ds(j * 16, 16)] = jnp.zeros((16,), jnp.float32)
                return 0

            lax.fori_loop(0, D // 16, _zcol, 0)
            return 0

        lax.fori_loop(0, ZR, _zrow, 0)

        ebase = wid * ET
        NG = ET // IG
        pltpu.async_copy(src_hbm.at[pl.ds(ebase, IG)], si_a, sem_s)
        pltpu.async_copy(dst_hbm.at[pl.ds(ebase, IG)], di_a, sem_d)
        pltpu.make_async_copy(src_hbm.at[pl.ds(ebase, IG)], si_a, sem_s).wait()
        pltpu.make_async_copy(dst_hbm.at[pl.ds(ebase, IG)], di_a, sem_d).wait()
        pltpu.async_copy(src_hbm.at[pl.ds(ebase + IG, IG)], si_b, sem_s)
        pltpu.async_copy(dst_hbm.at[pl.ds(ebase + IG, IG)], di_b, sem_d)
        pltpu.async_copy(y_hbm.at[si_a.at[0]], r0, sem0)

        def _zcp(r, _):
            pltpu.sync_copy(zb, acc.at[pl.ds(sid * NR + r * ZR, ZR)])
            return 0

        lax.fori_loop(0, NR // ZR, _zcp, 0)
        plsc.subcore_barrier()

        for g in range(NG):
            si, di = (si_a, di_a) if g % 2 == 0 else (si_b, di_b)
            sn, dn = (si_b, di_b) if g % 2 == 0 else (si_a, di_a)

            def _step(t, _, si=si, di=di):
                j = 2 * t
                pltpu.async_copy(y_hbm.at[si.at[j + 1]], r1, sem1)
                pltpu.make_async_copy(y_hbm.at[si.at[j]], r0, sem0).wait()
                pltpu.sync_copy(r0, acc.at[di.at[j]], add=True)
                pltpu.async_copy(y_hbm.at[si.at[j + 2]], r0, sem0)
                pltpu.make_async_copy(y_hbm.at[si.at[j + 1]], r1, sem1).wait()
                pltpu.sync_copy(r1, acc.at[di.at[j + 1]], add=True)
                return 0

            lax.fori_loop(0, IG // 2 - 1, _step, 0)
            pltpu.async_copy(y_hbm.at[si.at[IG - 1]], r1, sem1)
            pltpu.make_async_copy(y_hbm.at[si.at[IG - 2]], r0, sem0).wait()
            pltpu.sync_copy(r0, acc.at[di.at[IG - 2]], add=True)
            if g + 1 < NG:
                pltpu.make_async_copy(
                    src_hbm.at[pl.ds(ebase, IG)], sn, sem_s).wait()
                pltpu.make_async_copy(
                    dst_hbm.at[pl.ds(ebase, IG)], dn, sem_d).wait()
                pltpu.async_copy(y_hbm.at[sn.at[0]], r0, sem0)
            pltpu.make_async_copy(y_hbm.at[si.at[IG - 1]], r1, sem1).wait()
            pltpu.sync_copy(r1, acc.at[di.at[IG - 1]], add=True)
            if g + 2 < NG:
                off = ebase + (g + 2) * IG
                pltpu.async_copy(src_hbm.at[pl.ds(off, IG)], si, sem_s)
                pltpu.async_copy(dst_hbm.at[pl.ds(off, IG)], di, sem_d)

        plsc.subcore_barrier()
        pltpu.sync_copy(acc.at[pl.ds(sid * NR, NR)],
                        out_hbm.at[cid, pl.ds(sid * NR, NR)])

    return _agg


_agg128 = _make_agg(D1)


def _y1_body(x_ref, w_ref, g0_ref, g1_ref, o_ref):
    dinv = lax.rsqrt(g0_ref[...] + g1_ref[...] + 1.0)
    o_ref[...] = dinv * jnp.dot(x_ref[...], w_ref[...],
                                preferred_element_type=jnp.float32)


_y1 = pl.pallas_call(
    _y1_body,
    grid=(N // BM,),
    in_specs=[pl.BlockSpec((BM, D1), lambda i: (i, 0)),
              pl.BlockSpec((D1, D1), lambda i: (0, 0)),
              pl.BlockSpec((BM, 1), lambda i: (i, 0)),
              pl.BlockSpec((BM, 1), lambda i: (i, 0))],
    out_specs=pl.BlockSpec((BM, D1), lambda i: (i, 0)),
    out_shape=jax.ShapeDtypeStruct((N, D1), jnp.float32),
)


def _h_body(p_ref, y1_ref, g0_ref, g1_ref, b1_ref, o_ref):
    dinv = lax.rsqrt(g0_ref[...] + g1_ref[...] + 1.0)
    h = jnp.maximum(
        dinv * (p_ref[0] + p_ref[1] + y1_ref[...]) + b1_ref[...], 0.0)
    o_ref[...] = dinv * h


_h = pl.pallas_call(
    _h_body,
    grid=(N // BM,),
    in_specs=[pl.BlockSpec((NC, BM, D1), lambda i: (0, i, 0)),
              pl.BlockSpec((BM, D1), lambda i: (i, 0)),
              pl.BlockSpec((BM, 1), lambda i: (i, 0)),
              pl.BlockSpec((BM, 1), lambda i: (i, 0)),
              pl.BlockSpec((1, D1), lambda i: (0, 0))],
    out_specs=pl.BlockSpec((BM, D1), lambda i: (i, 0)),
    out_shape=jax.ShapeDtypeStruct((N, D1), jnp.float32),
)


def _z_body(q_ref, u_ref, g0_ref, g1_ref, b2_ref, w2_ref, o_ref):
    dinv = lax.rsqrt(g0_ref[...] + g1_ref[...] + 1.0)
    s = q_ref[0] + q_ref[1] + u_ref[...]
    o_ref[...] = dinv * jnp.dot(s, w2_ref[...],
                                preferred_element_type=jnp.float32) + b2_ref[...]


_z = pl.pallas_call(
    _z_body,
    grid=(N // BM,),
    in_specs=[pl.BlockSpec((NC, BM, D1), lambda i: (0, i, 0)),
              pl.BlockSpec((BM, D1), lambda i: (i, 0)),
              pl.BlockSpec((BM, 1), lambda i: (i, 0)),
              pl.BlockSpec((BM, 1), lambda i: (i, 0)),
              pl.BlockSpec((1, D2), lambda i: (0, 0)),
              pl.BlockSpec((D1, D2), lambda i: (0, 0))],
    out_specs=pl.BlockSpec((BM, D2), lambda i: (i, 0)),
    out_shape=jax.ShapeDtypeStruct((N, D2), jnp.float32),
)


def kernel(x, edge_index, W1, b1, W2, b2):
    ei = edge_index.astype(jnp.int32)
    k = jnp.arange(EP, dtype=jnp.int32)
    pad_s = k % N
    pad_d = (NPAD - 256) + (k % 256)
    src2 = jnp.concatenate([ei[0], pad_s]).reshape(EC, EW)
    dst2 = jnp.concatenate([ei[1], pad_d]).reshape(EC, EW)

    degp = _deg(dst2)
    g0 = degp[:NPAD, None]
    g1 = degp[NPAD:, None]
    y1 = _y1(x, W1, g0, g1)
    P = _agg128(y1, src2, dst2)
    u = _h(P, y1, g0, g1, b1[None, :])
    Q = _agg128(u, src2, dst2)
    return _z(Q, u, g0, g1, b2[None, :], W2)


# speedup vs baseline: 1.0039x; 1.0039x over previous
import functools

import jax
import jax.numpy as jnp
from jax import lax
from jax.experimental import pallas as pl
from jax.experimental.pallas import tpu as pltpu
from jax.experimental.pallas import tpu_sc as plsc

N = 10000
E = 320000
D1 = 128
D2 = 64
NC = 2
NS = 16
NW = NC * NS
NPAD = 10240
NR = NPAD // NS
EW = 128
EP = 7680
EC = (E + EP) // EW
ET = EC // NW
IG = 16
ZR = 32
BM = 1000

_mesh = plsc.VectorSubcoreMesh(core_axis_name="c", subcore_axis_name="s")


@functools.partial(
    pl.kernel,
    out_type=jax.ShapeDtypeStruct((NC * NPAD,), jnp.float32),
    mesh=_mesh,
    scratch_types=[
        pltpu.VMEM((ET, EW), jnp.int32),
        pltpu.VMEM((EW,), jnp.float32),
        pltpu.VMEM((NR,), jnp.float32),
        pltpu.SemaphoreType.DMA,
        pltpu.VMEM_SHARED((NPAD,), jnp.float32),
    ],
)
def _deg(dst_hbm, out_hbm, di_v, ones_v, zero_v, sem, acc):
    cid = lax.axis_index("c")
    sid = lax.axis_index("s")
    wid = cid * NS + sid

    pltpu.async_copy(dst_hbm.at[pl.ds(wid * ET, ET)], di_v, sem)

    def _fill(k, _):
        zero_v[pl.ds(k * 16, 16)] = jnp.zeros((16,), jnp.float32)
        return 0

    lax.fori_loop(0, NR // 16, _fill, 0)

    def _fill1(k, _):
        ones_v[pl.ds(k * 16, 16)] = jnp.ones((16,), jnp.float32)
        return 0

    lax.fori_loop(0, EW // 16, _fill1, 0)
    pltpu.sync_copy(zero_v, acc.at[pl.ds(sid * NR, NR)])
    pltpu.make_async_copy(dst_hbm.at[pl.ds(wid * ET, ET)], di_v, sem).wait()
    plsc.subcore_barrier()

    def _scat(j, _):
        pltpu.sync_copy(ones_v, acc.at[di_v.at[j]], add=True)
        return 0

    lax.fori_loop(0, ET, _scat, 0)
    plsc.subcore_barrier()
    pltpu.sync_copy(acc.at[pl.ds(sid * NR, NR)],
                    out_hbm.at[pl.ds(cid * NPAD + sid * NR, NR)])


def _make_agg(D):
    @functools.partial(
        pl.kernel,
        out_type=jax.ShapeDtypeStruct((NC, NPAD, D), jnp.float32),
        mesh=_mesh,
        scratch_types=[
            pltpu.VMEM((IG, EW), jnp.int32),
            pltpu.VMEM((IG, EW), jnp.int32),
            pltpu.VMEM((IG, EW), jnp.int32),
            pltpu.VMEM((IG, EW), jnp.int32),
            pltpu.VMEM((EW, D), jnp.float32),
            pltpu.VMEM((EW, D), jnp.float32),
            pltpu.VMEM((ZR, D), jnp.float32),
            pltpu.SemaphoreType.DMA,
            pltpu.SemaphoreType.DMA,
            pltpu.SemaphoreType.DMA,
            pltpu.SemaphoreType.DMA,
            pltpu.VMEM_SHARED((NPAD, D), jnp.float32),
        ],
    )
    def _agg(y_hbm, src_hbm, dst_hbm, out_hbm,
             si_a, si_b, di_a, di_b, r0, r1, zb,
             sem0, sem1, sem_s, sem_d, acc):
        cid = lax.axis_index("c")
        sid = lax.axis_index("s")
        wid = cid * NS + sid

        def _zrow(i, _):
            def _zcol(j, _):
                zb[i, pl.ds(j * 16, 16)] = jnp.zeros((16,), jnp.float32)
                return 0

            lax.fori_loop(0, D // 16, _zcol, 0)
            return 0

        lax.fori_loop(0, ZR, _zrow, 0)

        ebase = wid * ET
        NG = ET // IG
        pltpu.async_copy(src_hbm.at[pl.ds(ebase, IG)], si_a, sem_s)
        pltpu.async_copy(dst_hbm.at[pl.ds(ebase, IG)], di_a, sem_d)
        pltpu.make_async_copy(src_hbm.at[pl.ds(ebase, IG)], si_a, sem_s).wait()
        pltpu.make_async_copy(dst_hbm.at[pl.ds(ebase, IG)], di_a, sem_d).wait()
        pltpu.async_copy(src_hbm.at[pl.ds(ebase + IG, IG)], si_b, sem_s)
        pltpu.async_copy(dst_hbm.at[pl.ds(ebase + IG, IG)], di_b, sem_d)
        pltpu.async_copy(y_hbm.at[si_a.at[0]], r0, sem0)

        def _zcp(r, _):
            pltpu.sync_copy(zb, acc.at[pl.ds(sid * NR + r * ZR, ZR)])
            return 0

        lax.fori_loop(0, NR // ZR, _zcp, 0)
        plsc.subcore_barrier()

        for g in range(NG):
            si, di = (si_a, di_a) if g % 2 == 0 else (si_b, di_b)
            sn, dn = (si_b, di_b) if g % 2 == 0 else (si_a, di_a)

            def _step(t, _, si=si, di=di):
                j = 2 * t
                pltpu.async_copy(y_hbm.at[si.at[j + 1]], r1, sem1)
                pltpu.make_async_copy(y_hbm.at[si.at[j]], r0, sem0).wait()
                pltpu.sync_copy(r0, acc.at[di.at[j]], add=True)
                pltpu.async_copy(y_hbm.at[si.at[j + 2]], r0, sem0)
                pltpu.make_async_copy(y_hbm.at[si.at[j + 1]], r1, sem1).wait()
                pltpu.sync_copy(r1, acc.at[di.at[j + 1]], add=True)
                return 0

            lax.fori_loop(0, IG // 2 - 1, _step, 0)
            pltpu.async_copy(y_hbm.at[si.at[IG - 1]], r1, sem1)
            pltpu.make_async_copy(y_hbm.at[si.at[IG - 2]], r0, sem0).wait()
            pltpu.sync_copy(r0, acc.at[di.at[IG - 2]], add=True)
            if g + 1 < NG:
                pltpu.make_async_copy(
                    src_hbm.at[pl.ds(ebase, IG)], sn, sem_s).wait()
                pltpu.make_async_copy(
                    dst_hbm.at[pl.ds(ebase, IG)], dn, sem_d).wait()
                pltpu.async_copy(y_hbm.at[sn.at[0]], r0, sem0)
            pltpu.make_async_copy(y_hbm.at[si.at[IG - 1]], r1, sem1).wait()
            pltpu.sync_copy(r1, acc.at[di.at[IG - 1]], add=True)
            if g + 2 < NG:
                off = ebase + (g + 2) * IG
                pltpu.async_copy(src_hbm.at[pl.ds(off, IG)], si, sem_s)
                pltpu.async_copy(dst_hbm.at[pl.ds(off, IG)], di, sem_d)

        plsc.subcore_barrier()
        pltpu.sync_copy(acc.at[pl.ds(sid * NR, NR)],
                        out_hbm.at[cid, pl.ds(sid * NR, NR)])

    return _agg


_agg128 = _make_agg(D1)


def _y1_body(x_ref, w_ref, g0_ref, g1_ref, o_ref):
    dinv = lax.rsqrt(g0_ref[...] + g1_ref[...] + 1.0)
    o_ref[...] = dinv * jnp.dot(x_ref[...], w_ref[...],
                                preferred_element_type=jnp.float32)


_y1 = pl.pallas_call(
    _y1_body,
    grid=(N // BM,),
    in_specs=[pl.BlockSpec((BM, D1), lambda i: (i, 0)),
              pl.BlockSpec((D1, D1), lambda i: (0, 0)),
              pl.BlockSpec((BM, 1), lambda i: (i, 0)),
              pl.BlockSpec((BM, 1), lambda i: (i, 0))],
    out_specs=pl.BlockSpec((BM, D1), lambda i: (i, 0)),
    out_shape=jax.ShapeDtypeStruct((N, D1), jnp.float32),
)


def _h_body(p_ref, y1_ref, g0_ref, g1_ref, b1_ref, o_ref):
    dinv = lax.rsqrt(g0_ref[...] + g1_ref[...] + 1.0)
    h = jnp.maximum(
        dinv * (p_ref[0] + p_ref[1] + y1_ref[...]) + b1_ref[...], 0.0)
    o_ref[...] = dinv * h


_h = pl.pallas_call(
    _h_body,
    grid=(N // BM,),
    in_specs=[pl.BlockSpec((NC, BM, D1), lambda i: (0, i, 0)),
              pl.BlockSpec((BM, D1), lambda i: (i, 0)),
              pl.BlockSpec((BM, 1), lambda i: (i, 0)),
              pl.BlockSpec((BM, 1), lambda i: (i, 0)),
              pl.BlockSpec((1, D1), lambda i: (0, 0))],
    out_specs=pl.BlockSpec((BM, D1), lambda i: (i, 0)),
    out_shape=jax.ShapeDtypeStruct((N, D1), jnp.float32),
)


def _z_body(q_ref, u_ref, g0_ref, g1_ref, b2_ref, w2_ref, o_ref):
    dinv = lax.rsqrt(g0_ref[...] + g1_ref[...] + 1.0)
    s = q_ref[0] + q_ref[1] + u_ref[...]
    o_ref[...] = dinv * jnp.dot(s, w2_ref[...],
                                preferred_element_type=jnp.float32) + b2_ref[...]


_z = pl.pallas_call(
    _z_body,
    grid=(N // BM,),
    in_specs=[pl.BlockSpec((NC, BM, D1), lambda i: (0, i, 0)),
              pl.BlockSpec((BM, D1), lambda i: (i, 0)),
              pl.BlockSpec((BM, 1), lambda i: (i, 0)),
              pl.BlockSpec((BM, 1), lambda i: (i, 0)),
              pl.BlockSpec((1, D2), lambda i: (0, 0)),
              pl.BlockSpec((D1, D2), lambda i: (0, 0))],
    out_specs=pl.BlockSpec((BM, D2), lambda i: (i, 0)),
    out_shape=jax.ShapeDtypeStruct((N, D2), jnp.float32),
)


def kernel(x, edge_index, W1, b1, W2, b2):
    ei = edge_index.astype(jnp.int32)
    k = jnp.arange(EP, dtype=jnp.int32)
    pad_s = k % N
    pad_d = N + (k % (NPAD - N))
    src2 = jnp.concatenate([ei[0], pad_s]).reshape(EC, EW)
    dst2 = jnp.concatenate([ei[1], pad_d]).reshape(EC, EW)

    degp = _deg(dst2)
    g0 = degp[:NPAD, None]
    g1 = degp[NPAD:, None]
    y1 = _y1(x, W1, g0, g1)
    P = _agg128(y1, src2, dst2)
    u = _h(P, y1, g0, g1, b1[None, :])
    Q = _agg128(u, src2, dst2)
    return _z(Q, u, g0, g1, b2[None, :], W2)


# speedup vs baseline: 1.0063x; 1.0023x over previous
import functools

import jax
import jax.numpy as jnp
from jax import lax
from jax.experimental import pallas as pl
from jax.experimental.pallas import tpu as pltpu
from jax.experimental.pallas import tpu_sc as plsc

N = 10000
E = 320000
D1 = 128
D2 = 64
NC = 2
NS = 16
NW = NC * NS
NPAD = 10240
NR = NPAD // NS
EW = 128
EP = 7680
EC = (E + EP) // EW
ET = EC // NW
IG = 16
ZR = 32
BM = 1000

_mesh = plsc.VectorSubcoreMesh(core_axis_name="c", subcore_axis_name="s")


@functools.partial(
    pl.kernel,
    out_type=jax.ShapeDtypeStruct((NC * NPAD,), jnp.float32),
    mesh=_mesh,
    scratch_types=[
        pltpu.VMEM((ET, EW), jnp.int32),
        pltpu.VMEM((EW,), jnp.float32),
        pltpu.VMEM((NR,), jnp.float32),
        pltpu.SemaphoreType.DMA,
        pltpu.VMEM_SHARED((NPAD,), jnp.float32),
    ],
)
def _deg(dst_hbm, out_hbm, di_v, ones_v, zero_v, sem, acc):
    cid = lax.axis_index("c")
    sid = lax.axis_index("s")
    wid = cid * NS + sid

    pltpu.async_copy(dst_hbm.at[pl.ds(wid * ET, ET)], di_v, sem)

    def _fill(k, _):
        zero_v[pl.ds(k * 16, 16)] = jnp.zeros((16,), jnp.float32)
        return 0

    lax.fori_loop(0, NR // 16, _fill, 0)

    def _fill1(k, _):
        ones_v[pl.ds(k * 16, 16)] = jnp.ones((16,), jnp.float32)
        return 0

    lax.fori_loop(0, EW // 16, _fill1, 0)
    pltpu.sync_copy(zero_v, acc.at[pl.ds(sid * NR, NR)])
    pltpu.make_async_copy(dst_hbm.at[pl.ds(wid * ET, ET)], di_v, sem).wait()
    plsc.subcore_barrier()

    def _scat(j, _):
        pltpu.sync_copy(ones_v, acc.at[di_v.at[j]], add=True)
        return 0

    lax.fori_loop(0, ET, _scat, 0)
    plsc.subcore_barrier()
    pltpu.sync_copy(acc.at[pl.ds(sid * NR, NR)],
                    out_hbm.at[pl.ds(cid * NPAD + sid * NR, NR)])


def _make_agg(D):
    @functools.partial(
        pl.kernel,
        out_type=jax.ShapeDtypeStruct((NC, NPAD, D), jnp.float32),
        mesh=_mesh,
        scratch_types=[
            pltpu.VMEM((IG, EW), jnp.int32),
            pltpu.VMEM((IG, EW), jnp.int32),
            pltpu.VMEM((IG, EW), jnp.int32),
            pltpu.VMEM((IG, EW), jnp.int32),
            pltpu.VMEM((EW, D), jnp.float32),
            pltpu.VMEM((EW, D), jnp.float32),
            pltpu.VMEM((ZR, D), jnp.float32),
            pltpu.SemaphoreType.DMA,
            pltpu.SemaphoreType.DMA,
            pltpu.SemaphoreType.DMA,
            pltpu.SemaphoreType.DMA,
            pltpu.VMEM_SHARED((NPAD, D), jnp.float32),
        ],
    )
    def _agg(y_hbm, src_hbm, dst_hbm, out_hbm,
             si_a, si_b, di_a, di_b, r0, r1, zb,
             sem0, sem1, sem_s, sem_d, acc):
        cid = lax.axis_index("c")
        sid = lax.axis_index("s")
        wid = cid * NS + sid

        def _zrow(i, _):
            def _zcol(j, _):
                zb[i, pl.ds(j * 16, 16)] = jnp.zeros((16,), jnp.float32)
                return 0

            lax.fori_loop(0, D // 16, _zcol, 0)
            return 0

        lax.fori_loop(0, ZR, _zrow, 0)

        ebase = wid * ET
        NG = ET // IG
        pltpu.async_copy(src_hbm.at[pl.ds(ebase, IG)], si_a, sem_s)
        pltpu.async_copy(dst_hbm.at[pl.ds(ebase, IG)], di_a, sem_d)
        pltpu.make_async_copy(src_hbm.at[pl.ds(ebase, IG)], si_a, sem_s).wait()
        pltpu.make_async_copy(dst_hbm.at[pl.ds(ebase, IG)], di_a, sem_d).wait()
        pltpu.async_copy(src_hbm.at[pl.ds(ebase + IG, IG)], si_b, sem_s)
        pltpu.async_copy(dst_hbm.at[pl.ds(ebase + IG, IG)], di_b, sem_d)
        pltpu.async_copy(y_hbm.at[si_a.at[0]], r0, sem0)

        def _zcp(r, _):
            pltpu.sync_copy(zb, acc.at[pl.ds(sid * NR + r * ZR, ZR)])
            return 0

        lax.fori_loop(0, NR // ZR, _zcp, 0)
        plsc.subcore_barrier()

        for g in range(NG):
            si, di = (si_a, di_a) if g % 2 == 0 else (si_b, di_b)
            sn, dn = (si_b, di_b) if g % 2 == 0 else (si_a, di_a)

            def _step(t, _, si=si, di=di):
                j = 2 * t
                pltpu.async_copy(y_hbm.at[si.at[j + 1]], r1, sem1)
                pltpu.make_async_copy(y_hbm.at[si.at[j]], r0, sem0).wait()
                pltpu.sync_copy(r0, acc.at[di.at[j]], add=True)
                pltpu.async_copy(y_hbm.at[si.at[j + 2]], r0, sem0)
                pltpu.make_async_copy(y_hbm.at[si.at[j + 1]], r1, sem1).wait()
                pltpu.sync_copy(r1, acc.at[di.at[j + 1]], add=True)
                return 0

            lax.fori_loop(0, IG // 2 - 1, _step, 0)
            pltpu.async_copy(y_hbm.at[si.at[IG - 1]], r1, sem1)
            pltpu.make_async_copy(y_hbm.at[si.at[IG - 2]], r0, sem0).wait()
            pltpu.sync_copy(r0, acc.at[di.at[IG - 2]], add=True)
            if g + 1 < NG:
                pltpu.make_async_copy(
                    src_hbm.at[pl.ds(ebase, IG)], sn, sem_s).wait()
                pltpu.make_async_copy(
                    dst_hbm.at[pl.ds(ebase, IG)], dn, sem_d).wait()
                pltpu.async_copy(y_hbm.at[sn.at[0]], r0, sem0)
            pltpu.make_async_copy(y_hbm.at[si.at[IG - 1]], r1, sem1).wait()
            pltpu.sync_copy(r1, acc.at[di.at[IG - 1]], add=True)
            if g + 2 < NG:
                off = ebase + (g + 2) * IG
                pltpu.async_copy(src_hbm.at[pl.ds(off, IG)], si, sem_s)
                pltpu.async_copy(dst_hbm.at[pl.ds(off, IG)], di, sem_d)

        plsc.subcore_barrier()
        pltpu.sync_copy(acc.at[pl.ds(sid * NR, NR)],
                        out_hbm.at[cid, pl.ds(sid * NR, NR)])

    return _agg


_agg128 = _make_agg(D1)


def _mm_body(x_ref, w_ref, o_ref):
    o_ref[...] = jnp.dot(x_ref[...], w_ref[...],
                         preferred_element_type=jnp.float32)


_mm1 = pl.pallas_call(
    _mm_body,
    grid=(N // BM,),
    in_specs=[pl.BlockSpec((BM, D1), lambda i: (i, 0)),
              pl.BlockSpec((D1, D1), lambda i: (0, 0))],
    out_specs=pl.BlockSpec((BM, D1), lambda i: (i, 0)),
    out_shape=jax.ShapeDtypeStruct((N, D1), jnp.float32),
)


def _y1_body(xw_ref, g0_ref, g1_ref, o_ref):
    dinv = lax.rsqrt(g0_ref[...] + g1_ref[...] + 1.0)
    o_ref[...] = dinv * xw_ref[...]


_y1 = pl.pallas_call(
    _y1_body,
    grid=(N // BM,),
    in_specs=[pl.BlockSpec((BM, D1), lambda i: (i, 0)),
              pl.BlockSpec((BM, 1), lambda i: (i, 0)),
              pl.BlockSpec((BM, 1), lambda i: (i, 0))],
    out_specs=pl.BlockSpec((BM, D1), lambda i: (i, 0)),
    out_shape=jax.ShapeDtypeStruct((N, D1), jnp.float32),
)


def _h_body(p_ref, y1_ref, g0_ref, g1_ref, b1_ref, o_ref):
    dinv = lax.rsqrt(g0_ref[...] + g1_ref[...] + 1.0)
    h = jnp.maximum(
        dinv * (p_ref[0] + p_ref[1] + y1_ref[...]) + b1_ref[...], 0.0)
    o_ref[...] = dinv * h


_h = pl.pallas_call(
    _h_body,
    grid=(N // BM,),
    in_specs=[pl.BlockSpec((NC, BM, D1), lambda i: (0, i, 0)),
              pl.BlockSpec((BM, D1), lambda i: (i, 0)),
              pl.BlockSpec((BM, 1), lambda i: (i, 0)),
              pl.BlockSpec((BM, 1), lambda i: (i, 0)),
              pl.BlockSpec((1, D1), lambda i: (0, 0))],
    out_specs=pl.BlockSpec((BM, D1), lambda i: (i, 0)),
    out_shape=jax.ShapeDtypeStruct((N, D1), jnp.float32),
)


def _z_body(q_ref, u_ref, g0_ref, g1_ref, b2_ref, w2_ref, o_ref):
    dinv = lax.rsqrt(g0_ref[...] + g1_ref[...] + 1.0)
    s = q_ref[0] + q_ref[1] + u_ref[...]
    o_ref[...] = dinv * jnp.dot(s, w2_ref[...],
                                preferred_element_type=jnp.float32) + b2_ref[...]


_z = pl.pallas_call(
    _z_body,
    grid=(N // BM,),
    in_specs=[pl.BlockSpec((NC, BM, D1), lambda i: (0, i, 0)),
              pl.BlockSpec((BM, D1), lambda i: (i, 0)),
              pl.BlockSpec((BM, 1), lambda i: (i, 0)),
              pl.BlockSpec((BM, 1), lambda i: (i, 0)),
              pl.BlockSpec((1, D2), lambda i: (0, 0)),
              pl.BlockSpec((D1, D2), lambda i: (0, 0))],
    out_specs=pl.BlockSpec((BM, D2), lambda i: (i, 0)),
    out_shape=jax.ShapeDtypeStruct((N, D2), jnp.float32),
)


def kernel(x, edge_index, W1, b1, W2, b2):
    ei = edge_index.astype(jnp.int32)
    k = jnp.arange(EP, dtype=jnp.int32)
    pad_s = k % N
    pad_d = N + (k % (NPAD - N))
    src2 = jnp.concatenate([ei[0], pad_s]).reshape(EC, EW)
    dst2 = jnp.concatenate([ei[1], pad_d]).reshape(EC, EW)

    xw1 = _mm1(x, W1)
    degp = _deg(dst2)
    g0 = degp[:NPAD, None]
    g1 = degp[NPAD:, None]
    y1 = _y1(xw1, g0, g1)
    P = _agg128(y1, src2, dst2)
    u = _h(P, y1, g0, g1, b1[None, :])
    Q = _agg128(u, src2, dst2)
    return _z(Q, u, g0, g1, b2[None, :], W2)


# speedup vs baseline: 1.0268x; 1.0204x over previous
import functools

import jax
import jax.numpy as jnp
from jax import lax
from jax.experimental import pallas as pl
from jax.experimental.pallas import tpu as pltpu
from jax.experimental.pallas import tpu_sc as plsc

N = 10000
E = 320000
D1 = 128
D2 = 64
NC = 2
NS = 16
NW = NC * NS
NPAD = 10240
NR = NPAD // NS
EW = 128
EP = 7680
EC = (E + EP) // EW
ET = EC // NW
IG = 16
ZR = 32
BM = 1000

_mesh = plsc.VectorSubcoreMesh(core_axis_name="c", subcore_axis_name="s")


@functools.partial(
    pl.kernel,
    out_type=jax.ShapeDtypeStruct((NC * NPAD,), jnp.float32),
    mesh=_mesh,
    scratch_types=[
        pltpu.VMEM((ET, EW), jnp.int32),
        pltpu.VMEM((EW,), jnp.float32),
        pltpu.VMEM((NR,), jnp.float32),
        pltpu.SemaphoreType.DMA,
        pltpu.VMEM_SHARED((NPAD,), jnp.float32),
    ],
)
def _deg(dst_hbm, out_hbm, di_v, ones_v, zero_v, sem, acc):
    cid = lax.axis_index("c")
    sid = lax.axis_index("s")
    wid = cid * NS + sid

    pltpu.async_copy(dst_hbm.at[pl.ds(wid * ET, ET)], di_v, sem)

    def _fill(k, _):
        zero_v[pl.ds(k * 16, 16)] = jnp.zeros((16,), jnp.float32)
        return 0

    lax.fori_loop(0, NR // 16, _fill, 0)

    def _fill1(k, _):
        ones_v[pl.ds(k * 16, 16)] = jnp.ones((16,), jnp.float32)
        return 0

    lax.fori_loop(0, EW // 16, _fill1, 0)
    pltpu.sync_copy(zero_v, acc.at[pl.ds(sid * NR, NR)])
    pltpu.make_async_copy(dst_hbm.at[pl.ds(wid * ET, ET)], di_v, sem).wait()
    plsc.subcore_barrier()

    def _scat(j, _):
        pltpu.sync_copy(ones_v, acc.at[di_v.at[j]], add=True)
        return 0

    lax.fori_loop(0, ET, _scat, 0)
    plsc.subcore_barrier()
    pltpu.sync_copy(acc.at[pl.ds(sid * NR, NR)],
                    out_hbm.at[pl.ds(cid * NPAD + sid * NR, NR)])


def _make_agg(D):
    @functools.partial(
        pl.kernel,
        out_type=jax.ShapeDtypeStruct((NC, NPAD, D), jnp.float32),
        mesh=_mesh,
        scratch_types=[
            pltpu.VMEM((IG, EW), jnp.int32),
            pltpu.VMEM((IG, EW), jnp.int32),
            pltpu.VMEM((IG, EW), jnp.int32),
            pltpu.VMEM((IG, EW), jnp.int32),
            pltpu.VMEM((EW, D), jnp.float32),
            pltpu.VMEM((EW, D), jnp.float32),
            pltpu.VMEM((ZR, D), jnp.float32),
            pltpu.SemaphoreType.DMA,
            pltpu.SemaphoreType.DMA,
            pltpu.SemaphoreType.DMA,
            pltpu.SemaphoreType.DMA,
            pltpu.VMEM_SHARED((NPAD, D), jnp.float32),
        ],
    )
    def _agg(y_hbm, src_hbm, dst_hbm, out_hbm,
             si_a, si_b, di_a, di_b, r0, r1, zb,
             sem0, sem1, sem_s, sem_d, acc):
        cid = lax.axis_index("c")
        sid = lax.axis_index("s")
        wid = cid * NS + sid

        def _zrow(i, _):
            def _zcol(j, _):
                zb[i, pl.ds(j * 16, 16)] = jnp.zeros((16,), jnp.float32)
                return 0

            lax.fori_loop(0, D // 16, _zcol, 0)
            return 0

        lax.fori_loop(0, ZR, _zrow, 0)

        ebase = wid * ET
        NG = ET // IG
        pltpu.async_copy(src_hbm.at[pl.ds(ebase, IG)], si_a, sem_s)
        pltpu.async_copy(dst_hbm.at[pl.ds(ebase, IG)], di_a, sem_d)
        pltpu.make_async_copy(src_hbm.at[pl.ds(ebase, IG)], si_a, sem_s).wait()
        pltpu.make_async_copy(dst_hbm.at[pl.ds(ebase, IG)], di_a, sem_d).wait()
        pltpu.async_copy(src_hbm.at[pl.ds(ebase + IG, IG)], si_b, sem_s)
        pltpu.async_copy(dst_hbm.at[pl.ds(ebase + IG, IG)], di_b, sem_d)
        pltpu.async_copy(y_hbm.at[si_a.at[0]], r0, sem0)

        def _zcp(r, _):
            pltpu.sync_copy(zb, acc.at[pl.ds(sid * NR + r * ZR, ZR)])
            return 0

        lax.fori_loop(0, NR // ZR, _zcp, 0)
        plsc.subcore_barrier()

        for g in range(NG):
            si, di = (si_a, di_a) if g % 2 == 0 else (si_b, di_b)
            sn, dn = (si_b, di_b) if g % 2 == 0 else (si_a, di_a)

            def _step(t, _, si=si, di=di):
                j = 2 * t
                pltpu.async_copy(y_hbm.at[si.at[j + 1]], r1, sem1)
                pltpu.make_async_copy(y_hbm.at[si.at[j]], r0, sem0).wait()
                pltpu.sync_copy(r0, acc.at[di.at[j]], add=True)
                pltpu.async_copy(y_hbm.at[si.at[j + 2]], r0, sem0)
                pltpu.make_async_copy(y_hbm.at[si.at[j + 1]], r1, sem1).wait()
                pltpu.sync_copy(r1, acc.at[di.at[j + 1]], add=True)
                return 0

            lax.fori_loop(0, IG // 2 - 1, _step, 0)
            pltpu.async_copy(y_hbm.at[si.at[IG - 1]], r1, sem1)
            pltpu.make_async_copy(y_hbm.at[si.at[IG - 2]], r0, sem0).wait()
            pltpu.sync_copy(r0, acc.at[di.at[IG - 2]], add=True)
            if g + 1 < NG:
                pltpu.make_async_copy(
                    src_hbm.at[pl.ds(ebase, IG)], sn, sem_s).wait()
                pltpu.make_async_copy(
                    dst_hbm.at[pl.ds(ebase, IG)], dn, sem_d).wait()
                pltpu.async_copy(y_hbm.at[sn.at[0]], r0, sem0)
            pltpu.make_async_copy(y_hbm.at[si.at[IG - 1]], r1, sem1).wait()
            pltpu.sync_copy(r1, acc.at[di.at[IG - 1]], add=True)
            if g + 2 < NG:
                off = ebase + (g + 2) * IG
                pltpu.async_copy(src_hbm.at[pl.ds(off, IG)], si, sem_s)
                pltpu.async_copy(dst_hbm.at[pl.ds(off, IG)], di, sem_d)

        plsc.subcore_barrier()
        pltpu.sync_copy(acc.at[pl.ds(sid * NR, NR)],
                        out_hbm.at[cid, pl.ds(sid * NR, NR)])

    return _agg


_agg128 = _make_agg(D1)


def _mm_body(x_ref, w_ref, o_ref):
    o_ref[...] = jnp.dot(x_ref[...], w_ref[...],
                         preferred_element_type=jnp.float32)


_mm1 = pl.pallas_call(
    _mm_body,
    grid=(N // BM,),
    in_specs=[pl.BlockSpec((BM, D1), lambda i: (i, 0)),
              pl.BlockSpec((D1, D1), lambda i: (0, 0))],
    out_specs=pl.BlockSpec((BM, D1), lambda i: (i, 0)),
    out_shape=jax.ShapeDtypeStruct((N, D1), jnp.float32),
)


def _y1_body(xw_ref, g0_ref, g1_ref, o_ref):
    dinv = lax.rsqrt(g0_ref[...] + g1_ref[...] + 1.0)
    o_ref[...] = dinv * xw_ref[...]


_y1 = pl.pallas_call(
    _y1_body,
    grid=(N // BM,),
    in_specs=[pl.BlockSpec((BM, D1), lambda i: (i, 0)),
              pl.BlockSpec((BM, 1), lambda i: (i, 0)),
              pl.BlockSpec((BM, 1), lambda i: (i, 0))],
    out_specs=pl.BlockSpec((BM, D1), lambda i: (i, 0)),
    out_shape=jax.ShapeDtypeStruct((N, D1), jnp.float32),
)


def _h_body(p_ref, y1_ref, g0_ref, g1_ref, b1_ref, o_ref):
    dinv = lax.rsqrt(g0_ref[...] + g1_ref[...] + 1.0)
    h = jnp.maximum(
        dinv * (p_ref[0] + p_ref[1] + y1_ref[...]) + b1_ref[...], 0.0)
    o_ref[...] = dinv * h


_h = pl.pallas_call(
    _h_body,
    grid=(N // BM,),
    in_specs=[pl.BlockSpec((NC, BM, D1), lambda i: (0, i, 0)),
              pl.BlockSpec((BM, D1), lambda i: (i, 0)),
              pl.BlockSpec((BM, 1), lambda i: (i, 0)),
              pl.BlockSpec((BM, 1), lambda i: (i, 0)),
              pl.BlockSpec((1, D1), lambda i: (0, 0))],
    out_specs=pl.BlockSpec((BM, D1), lambda i: (i, 0)),
    out_shape=jax.ShapeDtypeStruct((N, D1), jnp.float32),
)


BMZ = 1024


def _z_body(q_ref, u_ref, g0_ref, g1_ref, b2_ref, w2_ref, o_ref):
    dinv = lax.rsqrt(g0_ref[...] + g1_ref[...] + 1.0)
    drow = dinv.reshape(1, BMZ)
    s = q_ref[0] + q_ref[1] + u_ref[...]
    zt = lax.dot_general(w2_ref[...], s, (((0,), (1,)), ((), ())),
                         preferred_element_type=jnp.float32)
    o_ref[...] = drow * zt + b2_ref[...]


_z = pl.pallas_call(
    _z_body,
    grid=(NPAD // BMZ,),
    in_specs=[pl.BlockSpec((NC, BMZ, D1), lambda i: (0, i, 0)),
              pl.BlockSpec((BMZ, D1), lambda i: (i, 0)),
              pl.BlockSpec((BMZ // EW, EW), lambda i: (i, 0)),
              pl.BlockSpec((BMZ // EW, EW), lambda i: (i, 0)),
              pl.BlockSpec((D2, 1), lambda i: (0, 0)),
              pl.BlockSpec((D1, D2), lambda i: (0, 0))],
    out_specs=pl.BlockSpec((D2, BMZ), lambda i: (0, i)),
    out_shape=jax.ShapeDtypeStruct((D2, N), jnp.float32),
)


def kernel(x, edge_index, W1, b1, W2, b2):
    ei = edge_index.astype(jnp.int32)
    k = jnp.arange(EP, dtype=jnp.int32)
    pad_s = k % N
    pad_d = N + (k % (NPAD - N))
    src2 = jnp.concatenate([ei[0], pad_s]).reshape(EC, EW)
    dst2 = jnp.concatenate([ei[1], pad_d]).reshape(EC, EW)

    xw1 = _mm1(x, W1)
    degp = _deg(dst2)
    g0 = degp[:NPAD, None]
    g1 = degp[NPAD:, None]
    y1 = _y1(xw1, g0, g1)
    P = _agg128(y1, src2, dst2)
    u = _h(P, y1, g0, g1, b1[None, :])
    Q = _agg128(u, src2, dst2)
    g0r = degp[:NPAD].reshape(NPAD // EW, EW)
    g1r = degp[NPAD:].reshape(NPAD // EW, EW)
    zt = _z(Q, u, g0r, g1r, b2[:, None], W2)
    return zt.T


# speedup vs baseline: 1.0431x; 1.0159x over previous
import functools

import jax
import jax.numpy as jnp
from jax import lax
from jax.experimental import pallas as pl
from jax.experimental.pallas import tpu as pltpu
from jax.experimental.pallas import tpu_sc as plsc

N = 10000
E = 320000
D1 = 128
D2 = 64
NC = 2
NS = 16
NW = NC * NS
NPAD = 10240
NR = NPAD // NS
EW = 128
EC = E // EW
ET = 76
EXC = EC - NW * ET
ZR = 32
BM = 1000

_mesh = plsc.VectorSubcoreMesh(core_axis_name="c", subcore_axis_name="s")


@functools.partial(
    pl.kernel,
    out_type=jax.ShapeDtypeStruct((NC * NPAD,), jnp.float32),
    mesh=_mesh,
    scratch_types=[
        pltpu.VMEM((2, EW), jnp.int32),
        pltpu.VMEM((2, EW), jnp.int32),
        pltpu.VMEM((2, EW), jnp.int32),
        pltpu.VMEM((2, EW), jnp.int32),
        pltpu.VMEM((EW,), jnp.float32),
        pltpu.VMEM((NR,), jnp.float32),
        pltpu.SemaphoreType.DMA,
        pltpu.VMEM_SHARED((NPAD,), jnp.float32),
    ],
)
def _deg(ei_hbm, out_hbm, a0, a1, b0, b1, ones_v, zero_v, sem, acc):
    cid = lax.axis_index("c")
    sid = lax.axis_index("s")
    wid = cid * NS + sid
    cb = wid * ET

    def _ld(c, buf):
        pltpu.async_copy(ei_hbm.at[:, pl.ds(c * EW, EW)], buf, sem)

    def _wt(buf):
        pltpu.make_async_copy(ei_hbm.at[:, pl.ds(0, EW)], buf, sem).wait()

    _ld(cb, a0)
    _ld(cb + 1, a1)

    def _fill(k, _):
        zero_v[pl.ds(k * 16, 16)] = jnp.zeros((16,), jnp.float32)
        return 0

    lax.fori_loop(0, NR // 16, _fill, 0)

    def _fill1(k, _):
        ones_v[pl.ds(k * 16, 16)] = jnp.ones((16,), jnp.float32)
        return 0

    lax.fori_loop(0, EW // 16, _fill1, 0)
    pltpu.sync_copy(zero_v, acc.at[pl.ds(sid * NR, NR)])
    _wt(a0)
    _wt(a1)
    _ld(cb + 2, b0)
    _ld(cb + 3, b1)
    plsc.subcore_barrier()

    def _sc(buf):
        pltpu.sync_copy(ones_v, acc.at[buf.at[1]], add=True)

    def _quad(q, _):
        c = cb + 4 * q
        _sc(a0)
        _sc(a1)

        @pl.when(q < ET // 4 - 1)
        def _pa():
            _ld(c + 4, a0)
            _ld(c + 5, a1)

        _wt(b0)
        _wt(b1)
        _sc(b0)
        _sc(b1)

        @pl.when(q < ET // 4 - 1)
        def _pb():
            _ld(c + 6, b0)
            _ld(c + 7, b1)
            _wt(a0)
            _wt(a1)

        return 0

    lax.fori_loop(0, ET // 4, _quad, 0)

    def _extra(k, _):
        c = NW * ET + wid + NW * k

        @pl.when(c < EC)
        def _do():
            _ld(c, a0)
            _wt(a0)
            _sc(a0)

        return 0

    lax.fori_loop(0, (EXC + NW - 1) // NW, _extra, 0)
    plsc.subcore_barrier()
    pltpu.sync_copy(acc.at[pl.ds(sid * NR, NR)],
                    out_hbm.at[pl.ds(cid * NPAD + sid * NR, NR)])


def _make_agg(D):
    @functools.partial(
        pl.kernel,
        out_type=jax.ShapeDtypeStruct((NC, NPAD, D), jnp.float32),
        mesh=_mesh,
        scratch_types=[
            pltpu.VMEM((2, EW), jnp.int32),
            pltpu.VMEM((2, EW), jnp.int32),
            pltpu.VMEM((2, EW), jnp.int32),
            pltpu.VMEM((2, EW), jnp.int32),
            pltpu.VMEM((EW, D), jnp.float32),
            pltpu.VMEM((EW, D), jnp.float32),
            pltpu.VMEM((ZR, D), jnp.float32),
            pltpu.SemaphoreType.DMA,
            pltpu.SemaphoreType.DMA,
            pltpu.SemaphoreType.DMA,
            pltpu.VMEM_SHARED((NPAD, D), jnp.float32),
        ],
    )
    def _agg(y_hbm, ei_hbm, out_hbm,
             a0, a1, b0, b1, r0, r1, zb, sem0, sem1, sem_i, acc):
        cid = lax.axis_index("c")
        sid = lax.axis_index("s")
        wid = cid * NS + sid
        cb = wid * ET

        def _ld(c, buf):
            pltpu.async_copy(ei_hbm.at[:, pl.ds(c * EW, EW)], buf, sem_i)

        def _wt(buf):
            pltpu.make_async_copy(ei_hbm.at[:, pl.ds(0, EW)], buf, sem_i).wait()

        _ld(cb, a0)
        _ld(cb + 1, a1)

        def _zrow(i, _):
            def _zcol(j, _):
                zb[i, pl.ds(j * 16, 16)] = jnp.zeros((16,), jnp.float32)
                return 0

            lax.fori_loop(0, D // 16, _zcol, 0)
            return 0

        lax.fori_loop(0, ZR, _zrow, 0)
        _wt(a0)
        _wt(a1)
        _ld(cb + 2, b0)
        _ld(cb + 3, b1)
        pltpu.async_copy(y_hbm.at[a0.at[0]], r0, sem0)

        def _zcp(r, _):
            pltpu.sync_copy(zb, acc.at[pl.ds(sid * NR + r * ZR, ZR)])
            return 0

        lax.fori_loop(0, NR // ZR, _zcp, 0)
        plsc.subcore_barrier()

        NQ = ET // 4

        def _quad(q, _):
            c = cb + 4 * q
            pltpu.async_copy(y_hbm.at[a1.at[0]], r1, sem1)
            pltpu.make_async_copy(y_hbm.at[a1.at[0]], r0, sem0).wait()
            pltpu.sync_copy(r0, acc.at[a0.at[1]], add=True)
            _wt(b0)
            _wt(b1)
            pltpu.async_copy(y_hbm.at[b0.at[0]], r0, sem0)
            pltpu.make_async_copy(y_hbm.at[a1.at[0]], r1, sem1).wait()
            pltpu.sync_copy(r1, acc.at[a1.at[1]], add=True)

            @pl.when(q < NQ - 1)
            def _pa():
                _ld(c + 4, a0)
                _ld(c + 5, a1)

            pltpu.async_copy(y_hbm.at[b1.at[0]], r1, sem1)
            pltpu.make_async_copy(y_hbm.at[b1.at[0]], r0, sem0).wait()
            pltpu.sync_copy(r0, acc.at[b0.at[1]], add=True)

            @pl.when(q < NQ - 1)
            def _pb():
                _wt(a0)
                _wt(a1)
                pltpu.async_copy(y_hbm.at[a0.at[0]], r0, sem0)

            pltpu.make_async_copy(y_hbm.at[b1.at[0]], r1, sem1).wait()
            pltpu.sync_copy(r1, acc.at[b1.at[1]], add=True)

            @pl.when(q < NQ - 1)
            def _pb2():
                _ld(c + 6, b0)
                _ld(c + 7, b1)

            return 0

        lax.fori_loop(0, NQ, _quad, 0)

        def _extra(k, _):
            c = NW * ET + wid + NW * k

            @pl.when(c < EC)
            def _do():
                _ld(c, a0)
                _wt(a0)
                pltpu.async_copy(y_hbm.at[a0.at[0]], r0, sem0)
                pltpu.make_async_copy(y_hbm.at[a0.at[0]], r0, sem0).wait()
                pltpu.sync_copy(r0, acc.at[a0.at[1]], add=True)

            return 0

        lax.fori_loop(0, (EXC + NW - 1) // NW, _extra, 0)
        plsc.subcore_barrier()
        pltpu.sync_copy(acc.at[pl.ds(sid * NR, NR)],
                        out_hbm.at[cid, pl.ds(sid * NR, NR)])

    return _agg


_agg128 = _make_agg(D1)


def _mm_body(x_ref, w_ref, o_ref):
    o_ref[...] = jnp.dot(x_ref[...], w_ref[...],
                         preferred_element_type=jnp.float32)


_mm1 = pl.pallas_call(
    _mm_body,
    grid=(N // BM,),
    in_specs=[pl.BlockSpec((BM, D1), lambda i: (i, 0)),
              pl.BlockSpec((D1, D1), lambda i: (0, 0))],
    out_specs=pl.BlockSpec((BM, D1), lambda i: (i, 0)),
    out_shape=jax.ShapeDtypeStruct((N, D1), jnp.float32),
)


def _y1_body(xw_ref, g0_ref, g1_ref, o_ref):
    dinv = lax.rsqrt(g0_ref[...] + g1_ref[...] + 1.0)
    o_ref[...] = dinv * xw_ref[...]


_y1 = pl.pallas_call(
    _y1_body,
    grid=(N // BM,),
    in_specs=[pl.BlockSpec((BM, D1), lambda i: (i, 0)),
              pl.BlockSpec((BM, 1), lambda i: (i, 0)),
              pl.BlockSpec((BM, 1), lambda i: (i, 0))],
    out_specs=pl.BlockSpec((BM, D1), lambda i: (i, 0)),
    out_shape=jax.ShapeDtypeStruct((N, D1), jnp.float32),
)


def _h_body(p_ref, y1_ref, g0_ref, g1_ref, b1_ref, o_ref):
    dinv = lax.rsqrt(g0_ref[...] + g1_ref[...] + 1.0)
    h = jnp.maximum(
        dinv * (p_ref[0] + p_ref[1] + y1_ref[...]) + b1_ref[...], 0.0)
    o_ref[...] = dinv * h


_h = pl.pallas_call(
    _h_body,
    grid=(N // BM,),
    in_specs=[pl.BlockSpec((NC, BM, D1), lambda i: (0, i, 0)),
              pl.BlockSpec((BM, D1), lambda i: (i, 0)),
              pl.BlockSpec((BM, 1), lambda i: (i, 0)),
              pl.BlockSpec((BM, 1), lambda i: (i, 0)),
              pl.BlockSpec((1, D1), lambda i: (0, 0))],
    out_specs=pl.BlockSpec((BM, D1), lambda i: (i, 0)),
    out_shape=jax.ShapeDtypeStruct((N, D1), jnp.float32),
)


BMZ = 1024


def _z_body(q_ref, u_ref, g0_ref, g1_ref, b2_ref, w2_ref, o_ref):
    dinv = lax.rsqrt(g0_ref[...] + g1_ref[...] + 1.0)
    drow = dinv.reshape(1, BMZ)
    s = q_ref[0] + q_ref[1] + u_ref[...]
    zt = lax.dot_general(w2_ref[...], s, (((0,), (1,)), ((), ())),
                         preferred_element_type=jnp.float32)
    o_ref[...] = drow * zt + b2_ref[...]


_z = pl.pallas_call(
    _z_body,
    grid=(NPAD // BMZ,),
    in_specs=[pl.BlockSpec((NC, BMZ, D1), lambda i: (0, i, 0)),
              pl.BlockSpec((BMZ, D1), lambda i: (i, 0)),
              pl.BlockSpec((BMZ // EW, EW), lambda i: (i, 0)),
              pl.BlockSpec((BMZ // EW, EW), lambda i: (i, 0)),
              pl.BlockSpec((D2, 1), lambda i: (0, 0)),
              pl.BlockSpec((D1, D2), lambda i: (0, 0))],
    out_specs=pl.BlockSpec((D2, BMZ), lambda i: (0, i)),
    out_shape=jax.ShapeDtypeStruct((D2, N), jnp.float32),
)


def kernel(x, edge_index, W1, b1, W2, b2):
    ei = edge_index.astype(jnp.int32)

    xw1 = _mm1(x, W1)
    degp = _deg(ei)
    g0 = degp[:NPAD, None]
    g1 = degp[NPAD:, None]
    y1 = _y1(xw1, g0, g1)
    P = _agg128(y1, ei)
    u = _h(P, y1, g0, g1, b1[None, :])
    Q = _agg128(u, ei)
    g0r = degp[:NPAD].reshape(NPAD // EW, EW)
    g1r = degp[NPAD:].reshape(NPAD // EW, EW)
    zt = _z(Q, u, g0r, g1r, b2[:, None], W2)
    return zt.T
